# baseline (speedup 1.0000x reference)
import functools
import math

import jax
import jax.numpy as jnp
from jax import lax
from jax.experimental import pallas as pl
from jax.experimental.pallas import tpu as pltpu
from jax.experimental.pallas import tpu_sc as plsc

D_MODEL = 64
SCALE = math.sqrt(D_MODEL)

NUM_CORES = 2
NUM_SUBCORES = 16
NUM_WORKERS = NUM_CORES * NUM_SUBCORES
CHUNK = 256
NBUF = 4
LANES = 16
ROWS_PER_IT = 4


def _embed_lookup(idx3d, table):
    n_chunks = idx3d.shape[1]
    b_per_w = n_chunks * CHUNK
    n_idx = NUM_WORKERS * b_per_w
    n_groups = n_chunks // NBUF
    mesh = plsc.VectorSubcoreMesh(core_axis_name="c", subcore_axis_name="s")

    @functools.partial(
        pl.kernel,
        mesh=mesh,
        out_type=jax.ShapeDtypeStruct((n_idx, D_MODEL), jnp.float32),
        scratch_types=[
            pltpu.VMEM((n_chunks, CHUNK), jnp.int32),
            pltpu.VMEM((NBUF, CHUNK, D_MODEL), jnp.float32),
            [pltpu.SemaphoreType.DMA] * NBUF,
            pltpu.SemaphoreType.DMA,
        ],
        compiler_params=pltpu.CompilerParams(use_tc_tiling_on_sc=False),
    )
    def body(idx_hbm, table_hbm, out_hbm, idx_v, rows_v, sem_g, sem_s):
        wid = lax.axis_index("s") * NUM_CORES + lax.axis_index("c")
        base = wid * b_per_w
        pltpu.sync_copy(idx_hbm.at[wid], idx_v)

        def group_body(g, carry):
            c0 = g * NBUF
            gather_handles = [
                pltpu.async_copy(
                    table_hbm.at[idx_v.at[c0 + b]], rows_v.at[b], sem_g[b]
                )
                for b in range(NBUF)
            ]
            out_handles = []
            for b in range(NBUF):
                gather_handles[b].wait()

                def scale_body(r, c, b=b):
                    for k in range(ROWS_PER_IT):
                        for j in range(D_MODEL // LANES):
                            sl = pl.ds(j * LANES, LANES)
                            row = r * ROWS_PER_IT + k
                            rows_v[b, row, sl] = rows_v[b, row, sl] * SCALE
                    return c

                lax.fori_loop(0, CHUNK // ROWS_PER_IT, scale_body, 0)
                out_handles.append(
                    pltpu.async_copy(
                        rows_v.at[b],
                        out_hbm.at[pl.ds(base + (c0 + b) * CHUNK, CHUNK)],
                        sem_s,
                    )
                )
            for h in out_handles:
                h.wait()
            return carry

        lax.fori_loop(0, n_groups, group_body, 0)

    return body(idx3d, table)


def kernel(x, table):
    s0, s1 = x.shape
    n_idx = s0 * s1
    b_per_w = n_idx // NUM_WORKERS
    idx3d = x.reshape(NUM_WORKERS, b_per_w // CHUNK, CHUNK)
    out = _embed_lookup(idx3d, table)
    return out.reshape(s0, s1, D_MODEL)
